# Initial kernel scaffold; baseline (speedup 1.0000x reference)
import jax
import jax.numpy as jnp
from jax.experimental import pallas as pl


def kernel(embeddings, peak_mask, peak_values):
    raise NotImplementedError("write your pallas kernel here")


# speedup vs baseline: 82.9956x; 82.9956x over previous
import functools

import jax
import jax.numpy as jnp
from jax.experimental import pallas as pl
from jax.experimental.pallas import tpu as pltpu

_TEMPERATURE = 0.1
_POSITIVE_MULTIPLES = (1, 2)
_POS_TOL_RATIO = 0.25
_POS_TOL_FRAMES = 2.0
_NEG_SAFE_RATIO = 0.5
_NEG_SAFE_FRAMES = 4.0
_NUM_NEGATIVES = 32
_MAX_ANCHORS = 64

_N = 4096
_D = 128
_A = _MAX_ANCHORS
_BIGP = 1.0e9
_GINF = 1.0e30
_NEGBIG = -1.0e30


def _loss_body(emb_ref, mask_ref, vals_ref, out_ref):
    b = pl.program_id(0)

    emb = emb_ref[0]
    mask = mask_ref[0]
    vals = vals_ref[0]

    act = mask > 0.0
    actf = act.astype(jnp.float32)
    n = jnp.sum(actf)

    iota = jax.lax.broadcasted_iota(jnp.int32, (1, _N), 1).astype(jnp.float32)
    kidx = jax.lax.broadcasted_iota(jnp.int32, (_A, 1), 0).astype(jnp.float32)

    keys0 = jnp.where(act, jnp.maximum(vals, 1e-6), -1.0)

    def pick(i, carry):
        keys, ta, wa = carry
        m = jnp.max(keys, axis=1, keepdims=True)
        idx = jnp.min(jnp.where(keys == m, iota, jnp.float32(_N)),
                      axis=1, keepdims=True)
        sel = kidx == i.astype(jnp.float32)
        ta = jnp.where(sel, idx, ta)
        wa = jnp.where(sel, m, wa)
        keys = jnp.where(iota == idx, -1.0, keys)
        return keys, ta, wa

    _, ta, wa = jax.lax.fori_loop(
        0, _A, pick,
        (keys0, jnp.zeros((_A, 1), jnp.float32),
         jnp.full((_A, 1), -1.0, jnp.float32)))

    def mmin(cond):
        return jnp.min(jnp.where(cond, iota, _BIGP), axis=1, keepdims=True)

    def mmax(cond):
        return jnp.max(jnp.where(cond, iota, -_BIGP), axis=1, keepdims=True)

    p1 = mmin(act & (iota > ta))
    p2 = mmin(act & (iota > p1))
    p3 = mmin(act & (iota > p2))
    q1 = mmax(act & (iota < ta))
    q2 = mmax(act & (iota < q1))

    p1e = p1 < _BIGP * 0.5
    p2e = p2 < _BIGP * 0.5
    p3e = p3 < _BIGP * 0.5
    q1e = q1 > -_BIGP * 0.5
    q2e = q2 > -_BIGP * 0.5

    gaps = [
        (q1 - q2, q2e),
        (ta - q1, q1e),
        (p1 - ta, p1e),
        (p2 - p1, p2e),
        (p3 - p2, p3e),
    ]
    gv = [jnp.where(v, g, _GINF) for g, v in gaps]
    cnt = sum(v.astype(jnp.float32) for _, v in gaps)

    ksel = jnp.floor(jnp.maximum(cnt - 1.0, 0.0) * 0.5)
    tau = jnp.zeros((_A, 1), jnp.float32)
    for i in range(5):
        rank = jnp.zeros((_A, 1), jnp.float32)
        for j in range(5):
            if j == i:
                continue
            lt = (gv[j] < gv[i]).astype(jnp.float32)
            eq = (gv[j] == gv[i]).astype(jnp.float32)
            rank = rank + lt + (eq if j < i else 0.0)
        tau = tau + jnp.where(rank == ksel, gv[i], 0.0)
    tau_ok = (cnt > 0.0) & (tau > 0.0)

    dist = jnp.abs(iota - ta)
    notap = iota != ta
    posm = jnp.zeros((_A, _N), jnp.bool_)
    for m in _POSITIVE_MULTIPLES:
        target = tau * float(m)
        tol = jnp.maximum(_POS_TOL_FRAMES, target * _POS_TOL_RATIO)
        posm = posm | (jnp.abs(dist - target) <= tol)
    posm = posm & act & notap
    pos_count = jnp.sum(posm.astype(jnp.float32), axis=1, keepdims=True)

    safety = jnp.maximum(_NEG_SAFE_FRAMES, tau * _NEG_SAFE_RATIO)
    neg0 = (dist > safety) & act & notap & (~posm)
    has_neg = jnp.max(neg0.astype(jnp.float32), axis=1, keepdims=True) > 0.0
    fallback = act & notap & (~posm)
    negm = (neg0 & has_neg) | (fallback & jnp.logical_not(has_neg))
    neg_count = jnp.sum(negm.astype(jnp.float32), axis=1, keepdims=True)

    negf = negm.astype(jnp.float32)
    tri = (jax.lax.broadcasted_iota(jnp.int32, (128, 128), 0) <=
           jax.lax.broadcasted_iota(jnp.int32, (128, 128), 1)
           ).astype(jnp.float32)
    carry = jnp.zeros((_A, 1), jnp.float32)
    chunks = []
    for c in range(_N // 128):
        ch = jax.lax.slice(negf, (0, c * 128), (_A, (c + 1) * 128))
        inc = jax.lax.dot_general(
            ch, tri, (((1,), (0,)), ((), ())),
            preferred_element_type=jnp.float32) + carry
        chunks.append(inc)
        carry = jax.lax.slice(inc, (0, 127), (_A, 128))
    csum = jnp.concatenate(chunks, axis=1)
    negm = negm & (csum <= float(_NUM_NEGATIVES))

    nrm = jnp.sqrt(jnp.sum(emb * emb, axis=1, keepdims=True))
    nemb = emb / jnp.maximum(nrm, 1e-12)
    onehot = (iota == ta).astype(jnp.float32)
    avec = jax.lax.dot_general(
        onehot, nemb, (((1,), (0,)), ((), ())),
        preferred_element_type=jnp.float32,
        precision=jax.lax.Precision.HIGHEST)
    sims = jax.lax.dot_general(
        avec, nemb, (((1,), (1,)), ((), ())),
        preferred_element_type=jnp.float32,
        precision=jax.lax.Precision.HIGHEST) * (1.0 / _TEMPERATURE)

    smax = jnp.max(jnp.where(negm, sims, _NEGBIG), axis=1, keepdims=True)
    sumexp = jnp.sum(jnp.exp(jnp.where(negm, sims, _NEGBIG) - smax),
                     axis=1, keepdims=True)
    neg_lse = smax + jnp.log(sumexp)

    z = neg_lse - sims
    losses = jnp.maximum(z, 0.0) + jnp.log1p(jnp.exp(-jnp.abs(z)))
    anchor_loss = (jnp.sum(jnp.where(posm, losses, 0.0), axis=1, keepdims=True)
                   / jnp.maximum(pos_count, 1.0))

    ok = ((kidx < jnp.minimum(n, float(_A))) & (n >= 2.0) & tau_ok &
          (pos_count > 0.0) & (neg_count > 0.0))
    twl = jnp.sum(jnp.where(ok, wa * anchor_loss, 0.0))
    tw = jnp.sum(jnp.where(ok, wa, 0.0))

    lane = jax.lax.broadcasted_iota(jnp.int32, (1, 128), 1)
    row = (jnp.where(lane == 0, twl, 0.0) +
           jnp.where(lane == 1, tw, 0.0)).astype(jnp.float32)

    @pl.when(b == 0)
    def _():
        out_ref[...] = jnp.zeros_like(out_ref)

    out_ref[...] += row


@jax.jit
def kernel(embeddings, peak_mask, peak_values):
    B = peak_mask.shape[0]
    mask3 = peak_mask.reshape(B, 1, _N).astype(jnp.float32)
    vals3 = peak_values.reshape(B, 1, _N).astype(jnp.float32)

    out = pl.pallas_call(
        _loss_body,
        grid=(B,),
        in_specs=[
            pl.BlockSpec((1, _N, _D), lambda b: (b, 0, 0)),
            pl.BlockSpec((1, 1, _N), lambda b: (b, 0, 0)),
            pl.BlockSpec((1, 1, _N), lambda b: (b, 0, 0)),
        ],
        out_specs=pl.BlockSpec((1, 128), lambda b: (0, 0)),
        out_shape=jax.ShapeDtypeStruct((1, 128), jnp.float32),
        compiler_params=pltpu.CompilerParams(
            dimension_semantics=("arbitrary",)),
    )(embeddings, mask3, vals3)

    twl = out[0, 0]
    tw = out[0, 1]
    denom = jnp.where(tw > 0.0, tw, 1.0)
    return jnp.where(tw > 0.0, twl / denom, jnp.zeros(()))


# speedup vs baseline: 121.4610x; 1.4635x over previous
import functools

import jax
import jax.numpy as jnp
from jax.experimental import pallas as pl
from jax.experimental.pallas import tpu as pltpu

_TEMPERATURE = 0.1
_POSITIVE_MULTIPLES = (1, 2)
_POS_TOL_RATIO = 0.25
_POS_TOL_FRAMES = 2.0
_NEG_SAFE_RATIO = 0.5
_NEG_SAFE_FRAMES = 4.0
_NUM_NEGATIVES = 32
_MAX_ANCHORS = 64

_N = 4096
_D = 128
_A = _MAX_ANCHORS
_BIGP = 1.0e9
_GINF = 1.0e30
_NEGBIG = -1.0e30


def _loss_body(emb_ref, mask_ref, vals_ref, out_ref):
    b = pl.program_id(0)

    emb = emb_ref[0]
    mask = mask_ref[0]
    vals = vals_ref[0]

    act = mask > 0.0
    actf = act.astype(jnp.float32)
    n = jnp.sum(actf)

    iota = jax.lax.broadcasted_iota(jnp.int32, (1, _N), 1).astype(jnp.float32)
    kidx = jax.lax.broadcasted_iota(jnp.int32, (_A, 1), 0).astype(jnp.float32)

    keys = jnp.where(act, jnp.maximum(vals, 1e-6), 0.0)
    keys_i = jax.lax.bitcast_convert_type(keys, jnp.int32)

    def bisect(_, lohi):
        lo, hi = lohi
        mid = lo + (hi - lo) // 2
        c = jnp.sum((keys_i > mid).astype(jnp.int32))
        small = c < _A
        return jnp.where(small, lo, mid + 1), jnp.where(small, mid, hi)

    lo0 = jnp.int32(0)
    hi0 = jnp.int32(2147483647)
    _, thr = jax.lax.fori_loop(0, 31, bisect, (lo0, hi0))

    gtm = keys_i > thr
    tie = act & (keys_i == thr)
    n_gt = jnp.sum(gtm.astype(jnp.float32))

    nchunk = _N // 128
    mblk = (jax.lax.broadcasted_iota(jnp.int32, (_N, nchunk), 0) // 128 ==
            jax.lax.broadcasted_iota(jnp.int32, (_N, nchunk), 1)
            ).astype(jnp.float32)
    stri = (jax.lax.broadcasted_iota(jnp.int32, (nchunk, nchunk), 0) <
            jax.lax.broadcasted_iota(jnp.int32, (nchunk, nchunk), 1)
            ).astype(jnp.float32)
    tri = (jax.lax.broadcasted_iota(jnp.int32, (128, 128), 0) <=
           jax.lax.broadcasted_iota(jnp.int32, (128, 128), 1)
           ).astype(jnp.float32)

    def row_cumsum(x):
        r = x.shape[0]
        bsum = jax.lax.dot_general(
            x, mblk, (((1,), (0,)), ((), ())),
            preferred_element_type=jnp.float32)
        carry = jax.lax.dot_general(
            bsum, stri, (((1,), (0,)), ((), ())),
            preferred_element_type=jnp.float32)
        chunks = []
        for c in range(nchunk):
            ch = jax.lax.slice(x, (0, c * 128), (r, (c + 1) * 128))
            inc = jax.lax.dot_general(
                ch, tri, (((1,), (0,)), ((), ())),
                preferred_element_type=jnp.float32)
            chunks.append(inc + jax.lax.slice(carry, (0, c), (r, c + 1)))
        return jnp.concatenate(chunks, axis=1)

    tie_cum = row_cumsum(tie.astype(jnp.float32))
    selected = gtm | (tie & (tie_cum <= float(_A) - n_gt))
    self_f = selected.astype(jnp.float32)
    rank = row_cumsum(self_f) - self_f

    onehot = (selected & (rank == kidx)).astype(jnp.float32)
    ta = jax.lax.dot_general(
        onehot, iota, (((1,), (1,)), ((), ())),
        preferred_element_type=jnp.float32,
        precision=jax.lax.Precision.HIGHEST)
    wa = jax.lax.dot_general(
        onehot, keys, (((1,), (1,)), ((), ())),
        preferred_element_type=jnp.float32,
        precision=jax.lax.Precision.HIGHEST)

    def mmin(cond):
        return jnp.min(jnp.where(cond, iota, _BIGP), axis=1, keepdims=True)

    def mmax(cond):
        return jnp.max(jnp.where(cond, iota, -_BIGP), axis=1, keepdims=True)

    p1 = mmin(act & (iota > ta))
    p2 = mmin(act & (iota > p1))
    p3 = mmin(act & (iota > p2))
    q1 = mmax(act & (iota < ta))
    q2 = mmax(act & (iota < q1))

    p1e = p1 < _BIGP * 0.5
    p2e = p2 < _BIGP * 0.5
    p3e = p3 < _BIGP * 0.5
    q1e = q1 > -_BIGP * 0.5
    q2e = q2 > -_BIGP * 0.5

    gaps = [
        (q1 - q2, q2e),
        (ta - q1, q1e),
        (p1 - ta, p1e),
        (p2 - p1, p2e),
        (p3 - p2, p3e),
    ]
    gv = [jnp.where(v, g, _GINF) for g, v in gaps]
    cnt = sum(v.astype(jnp.float32) for _, v in gaps)

    ksel = jnp.floor(jnp.maximum(cnt - 1.0, 0.0) * 0.5)
    tau = jnp.zeros((_A, 1), jnp.float32)
    for i in range(5):
        rank = jnp.zeros((_A, 1), jnp.float32)
        for j in range(5):
            if j == i:
                continue
            lt = (gv[j] < gv[i]).astype(jnp.float32)
            eq = (gv[j] == gv[i]).astype(jnp.float32)
            rank = rank + lt + (eq if j < i else 0.0)
        tau = tau + jnp.where(rank == ksel, gv[i], 0.0)
    tau_ok = (cnt > 0.0) & (tau > 0.0)

    dist = jnp.abs(iota - ta)
    notap = iota != ta
    posm = jnp.zeros((_A, _N), jnp.bool_)
    for m in _POSITIVE_MULTIPLES:
        target = tau * float(m)
        tol = jnp.maximum(_POS_TOL_FRAMES, target * _POS_TOL_RATIO)
        posm = posm | (jnp.abs(dist - target) <= tol)
    posm = posm & act & notap
    pos_count = jnp.sum(posm.astype(jnp.float32), axis=1, keepdims=True)

    safety = jnp.maximum(_NEG_SAFE_FRAMES, tau * _NEG_SAFE_RATIO)
    neg0 = (dist > safety) & act & notap & (~posm)
    has_neg = jnp.max(neg0.astype(jnp.float32), axis=1, keepdims=True) > 0.0
    fallback = act & notap & (~posm)
    negm = (neg0 & has_neg) | (fallback & jnp.logical_not(has_neg))
    neg_count = jnp.sum(negm.astype(jnp.float32), axis=1, keepdims=True)

    csum = row_cumsum(negm.astype(jnp.float32))
    negm = negm & (csum <= float(_NUM_NEGATIVES))

    ss = jnp.sum(emb * emb, axis=1, keepdims=True)
    nemb = emb * jax.lax.rsqrt(jnp.maximum(ss, 1e-24))
    avec = jax.lax.dot_general(
        onehot, nemb, (((1,), (0,)), ((), ())),
        preferred_element_type=jnp.float32,
        precision=jax.lax.Precision.HIGHEST)
    sims = jax.lax.dot_general(
        avec, nemb, (((1,), (1,)), ((), ())),
        preferred_element_type=jnp.float32,
        precision=jax.lax.Precision.HIGHEST) * (1.0 / _TEMPERATURE)

    smax = 1.0 / _TEMPERATURE
    sumexp = jnp.sum(jnp.exp(jnp.where(negm, sims, _NEGBIG) - smax),
                     axis=1, keepdims=True)
    neg_lse = smax + jnp.log(sumexp)

    z = neg_lse - sims
    losses = jnp.maximum(z, 0.0) + jnp.log1p(jnp.exp(-jnp.abs(z)))
    anchor_loss = (jnp.sum(jnp.where(posm, losses, 0.0), axis=1, keepdims=True)
                   / jnp.maximum(pos_count, 1.0))

    ok = ((kidx < jnp.minimum(n, float(_A))) & (n >= 2.0) & tau_ok &
          (pos_count > 0.0) & (neg_count > 0.0))
    twl = jnp.sum(jnp.where(ok, wa * anchor_loss, 0.0))
    tw = jnp.sum(jnp.where(ok, wa, 0.0))

    lane = jax.lax.broadcasted_iota(jnp.int32, (1, 128), 1)
    row = (jnp.where(lane == 0, twl, 0.0) +
           jnp.where(lane == 1, tw, 0.0)).astype(jnp.float32)

    @pl.when(b == 0)
    def _():
        out_ref[...] = jnp.zeros_like(out_ref)

    out_ref[...] += row


@jax.jit
def kernel(embeddings, peak_mask, peak_values):
    B = peak_mask.shape[0]
    mask3 = peak_mask.reshape(B, 1, _N).astype(jnp.float32)
    vals3 = peak_values.reshape(B, 1, _N).astype(jnp.float32)

    out = pl.pallas_call(
        _loss_body,
        grid=(B,),
        in_specs=[
            pl.BlockSpec((1, _N, _D), lambda b: (b, 0, 0)),
            pl.BlockSpec((1, 1, _N), lambda b: (b, 0, 0)),
            pl.BlockSpec((1, 1, _N), lambda b: (b, 0, 0)),
        ],
        out_specs=pl.BlockSpec((1, 128), lambda b: (0, 0)),
        out_shape=jax.ShapeDtypeStruct((1, 128), jnp.float32),
        compiler_params=pltpu.CompilerParams(
            dimension_semantics=("arbitrary",)),
    )(embeddings, mask3, vals3)

    twl = out[0, 0]
    tw = out[0, 1]
    denom = jnp.where(tw > 0.0, tw, 1.0)
    return jnp.where(tw > 0.0, twl / denom, jnp.zeros(()))


# speedup vs baseline: 161.4423x; 1.3292x over previous
import functools

import jax
import jax.numpy as jnp
from jax.experimental import pallas as pl
from jax.experimental.pallas import tpu as pltpu

_TEMPERATURE = 0.1
_POSITIVE_MULTIPLES = (1, 2)
_POS_TOL_RATIO = 0.25
_POS_TOL_FRAMES = 2.0
_NEG_SAFE_RATIO = 0.5
_NEG_SAFE_FRAMES = 4.0
_NUM_NEGATIVES = 32
_MAX_ANCHORS = 64

_N = 4096
_D = 128
_A = _MAX_ANCHORS
_BIGP = 1.0e9
_GINF = 1.0e30
_NEGBIG = -1.0e30


def _loss_body(emb_ref, mask_ref, vals_ref, mblk_ref, stri_ref, tri_ref,
               out_ref):
    b = pl.program_id(0)

    emb = emb_ref[0]
    mask = mask_ref[0]
    vals = vals_ref[0]

    act = mask > 0.0
    actf = act.astype(jnp.float32)
    n = jnp.sum(actf)

    iota = jax.lax.broadcasted_iota(jnp.int32, (1, _N), 1).astype(jnp.float32)
    kidx = jax.lax.broadcasted_iota(jnp.int32, (_A, 1), 0).astype(jnp.float32)

    keys = jnp.where(act, jnp.maximum(vals, 1e-6), 0.0)
    keys_i = jax.lax.bitcast_convert_type(keys, jnp.int32)

    def bisect(_, lohi):
        lo, hi = lohi
        mid = lo + (hi - lo) // 2
        c = jnp.sum((keys_i > mid).astype(jnp.int32))
        small = c < _A
        return jnp.where(small, lo, mid + 1), jnp.where(small, mid, hi)

    lo0 = jnp.int32(0)
    hi0 = jnp.int32(2147483647)
    _, thr = jax.lax.fori_loop(0, 31, bisect, (lo0, hi0))

    gtm = keys_i > thr
    tie = act & (keys_i == thr)
    n_gt = jnp.sum(gtm.astype(jnp.float32))

    nchunk = _N // 128
    mblk = mblk_ref[...]
    stri = stri_ref[...]
    tri = tri_ref[...]

    def row_cumsum(x):
        r = x.shape[0]
        bsum = jax.lax.dot_general(
            x, mblk, (((1,), (0,)), ((), ())),
            preferred_element_type=jnp.float32)
        carry = jax.lax.dot_general(
            bsum, stri, (((1,), (0,)), ((), ())),
            preferred_element_type=jnp.float32)
        chunks = []
        for c in range(nchunk):
            ch = jax.lax.slice(x, (0, c * 128), (r, (c + 1) * 128))
            inc = jax.lax.dot_general(
                ch, tri, (((1,), (0,)), ((), ())),
                preferred_element_type=jnp.float32)
            chunks.append(inc + jax.lax.slice(carry, (0, c), (r, c + 1)))
        return jnp.concatenate(chunks, axis=1)

    both_cum = row_cumsum(jnp.concatenate(
        [gtm.astype(jnp.float32), tie.astype(jnp.float32)], axis=0))
    gtm_cum = jax.lax.slice(both_cum, (0, 0), (1, _N))
    tie_cum = jax.lax.slice(both_cum, (1, 0), (2, _N))
    slots_left = float(_A) - n_gt
    selected = gtm | (tie & (tie_cum <= slots_left))
    self_f = selected.astype(jnp.float32)
    rank = gtm_cum + jnp.minimum(tie_cum, slots_left) - self_f

    oh = selected & (rank == kidx)
    onehot = oh.astype(jnp.float32)
    ta = jnp.sum(jnp.where(oh, iota, 0.0), axis=1, keepdims=True)
    wa = jnp.sum(jnp.where(oh, keys, 0.0), axis=1, keepdims=True)

    def mmin(cond):
        return jnp.min(jnp.where(cond, iota, _BIGP), axis=1, keepdims=True)

    def mmax(cond):
        return jnp.max(jnp.where(cond, iota, -_BIGP), axis=1, keepdims=True)

    p1 = mmin(act & (iota > ta))
    p2 = mmin(act & (iota > p1))
    p3 = mmin(act & (iota > p2))
    q1 = mmax(act & (iota < ta))
    q2 = mmax(act & (iota < q1))

    p1e = p1 < _BIGP * 0.5
    p2e = p2 < _BIGP * 0.5
    p3e = p3 < _BIGP * 0.5
    q1e = q1 > -_BIGP * 0.5
    q2e = q2 > -_BIGP * 0.5

    gaps = [
        (q1 - q2, q2e),
        (ta - q1, q1e),
        (p1 - ta, p1e),
        (p2 - p1, p2e),
        (p3 - p2, p3e),
    ]
    gv = [jnp.where(v, g, _GINF) for g, v in gaps]
    cnt = sum(v.astype(jnp.float32) for _, v in gaps)

    ksel = jnp.floor(jnp.maximum(cnt - 1.0, 0.0) * 0.5)
    tau = jnp.zeros((_A, 1), jnp.float32)
    for i in range(5):
        rank = jnp.zeros((_A, 1), jnp.float32)
        for j in range(5):
            if j == i:
                continue
            lt = (gv[j] < gv[i]).astype(jnp.float32)
            eq = (gv[j] == gv[i]).astype(jnp.float32)
            rank = rank + lt + (eq if j < i else 0.0)
        tau = tau + jnp.where(rank == ksel, gv[i], 0.0)
    tau_ok = (cnt > 0.0) & (tau > 0.0)

    dist = jnp.abs(iota - ta)
    base = act & (iota != ta)
    posm = jnp.zeros((_A, _N), jnp.bool_)
    for m in _POSITIVE_MULTIPLES:
        target = tau * float(m)
        tol = jnp.maximum(_POS_TOL_FRAMES, target * _POS_TOL_RATIO)
        posm = posm | ((dist >= target - tol) & (dist <= target + tol))
    posm = posm & base
    pos_count = jnp.sum(posm.astype(jnp.float32), axis=1, keepdims=True)

    safety = jnp.maximum(_NEG_SAFE_FRAMES, tau * _NEG_SAFE_RATIO)
    nb = base & (~posm)
    far = dist > safety
    neg0 = nb & far
    has_neg = jnp.max(neg0.astype(jnp.float32), axis=1, keepdims=True) > 0.0
    negm = nb & (far | jnp.logical_not(has_neg))
    neg_count = jnp.sum(negm.astype(jnp.float32), axis=1, keepdims=True)

    csum = row_cumsum(negm.astype(jnp.float32))
    negm = negm & (csum <= float(_NUM_NEGATIVES))

    ss = jnp.sum(emb * emb, axis=1, keepdims=True)
    nemb = emb * jax.lax.rsqrt(jnp.maximum(ss, 1e-24))
    avec = jax.lax.dot_general(
        onehot, nemb, (((1,), (0,)), ((), ())),
        preferred_element_type=jnp.float32,
        precision=jax.lax.Precision.DEFAULT)
    sims = jax.lax.dot_general(
        avec, nemb, (((1,), (1,)), ((), ())),
        preferred_element_type=jnp.float32,
        precision=jax.lax.Precision.DEFAULT) * (1.0 / _TEMPERATURE)

    smax = 1.0 / _TEMPERATURE
    sumexp = jnp.sum(jnp.exp(jnp.where(negm, sims, _NEGBIG) - smax),
                     axis=1, keepdims=True)
    neg_lse = smax + jnp.log(sumexp)

    z = neg_lse - sims
    losses = jnp.log1p(jnp.exp(z))
    anchor_loss = (jnp.sum(jnp.where(posm, losses, 0.0), axis=1, keepdims=True)
                   / jnp.maximum(pos_count, 1.0))

    ok = ((kidx < jnp.minimum(n, float(_A))) & (n >= 2.0) & tau_ok &
          (pos_count > 0.0) & (neg_count > 0.0))
    twl = jnp.sum(jnp.where(ok, wa * anchor_loss, 0.0))
    tw = jnp.sum(jnp.where(ok, wa, 0.0))

    lane = jax.lax.broadcasted_iota(jnp.int32, (1, 128), 1)
    row = (jnp.where(lane == 0, twl, 0.0) +
           jnp.where(lane == 1, tw, 0.0)).astype(jnp.float32)

    @pl.when(b == 0)
    def _():
        out_ref[...] = jnp.zeros_like(out_ref)

    out_ref[...] += row


@jax.jit
def kernel(embeddings, peak_mask, peak_values):
    B = peak_mask.shape[0]
    nchunk = _N // 128
    mask3 = peak_mask.reshape(B, 1, _N).astype(jnp.float32)
    vals3 = peak_values.reshape(B, 1, _N).astype(jnp.float32)

    ri = jnp.arange(_N, dtype=jnp.int32)
    ci = jnp.arange(nchunk, dtype=jnp.int32)
    mblk = (ri[:, None] // 128 == ci[None, :]).astype(jnp.float32)
    stri = (ci[:, None] < ci[None, :]).astype(jnp.float32)
    k128 = jnp.arange(128, dtype=jnp.int32)
    tri = (k128[:, None] <= k128[None, :]).astype(jnp.float32)

    out = pl.pallas_call(
        _loss_body,
        grid=(B,),
        in_specs=[
            pl.BlockSpec((1, _N, _D), lambda b: (b, 0, 0)),
            pl.BlockSpec((1, 1, _N), lambda b: (b, 0, 0)),
            pl.BlockSpec((1, 1, _N), lambda b: (b, 0, 0)),
            pl.BlockSpec((_N, nchunk), lambda b: (0, 0)),
            pl.BlockSpec((nchunk, nchunk), lambda b: (0, 0)),
            pl.BlockSpec((128, 128), lambda b: (0, 0)),
        ],
        out_specs=pl.BlockSpec((1, 128), lambda b: (0, 0)),
        out_shape=jax.ShapeDtypeStruct((1, 128), jnp.float32),
        compiler_params=pltpu.CompilerParams(
            dimension_semantics=("arbitrary",)),
    )(embeddings, mask3, vals3, mblk, stri, tri)

    twl = out[0, 0]
    tw = out[0, 1]
    denom = jnp.where(tw > 0.0, tw, 1.0)
    return jnp.where(tw > 0.0, twl / denom, jnp.zeros(()))
